# Initial kernel scaffold; baseline (speedup 1.0000x reference)
import jax
import jax.numpy as jnp
from jax.experimental import pallas as pl


def kernel(x):
    raise NotImplementedError("write your pallas kernel here")


# speedup vs baseline: 1.0467x; 1.0467x over previous
import functools

import numpy as np
import jax
import jax.numpy as jnp
from jax import lax
from jax.experimental import pallas as pl
from jax.experimental.pallas import tpu as pltpu
from jax.experimental.pallas import tpu_sc as plsc

N = 64
NSCALE = 4
PLANE = NSCALE * N * (N + 1)
PLANE_PAD = PLANE + 16

def _toff(r, p):
    return 256 + (r * 6 + (p - 1)) * 80

W_SIZE = 256 + NSCALE * 6 * 80


def _build_static():
    tab = []
    for r in range(NSCALE):
        for p in range(1, (6 - r) + 1):
            length = 65 - (1 << p)
            prev = _toff(r, p - 1) if p > 1 else r * 64
            cur = _toff(r, p)
            h = 1 << (p - 1)
            for s0 in range(0, length, 16):
                tab.append((cur + s0, prev + s0, prev + s0 + h))
    vops = []
    for r in range(NSCALE):
        beta = 1 << r
        S = N >> r
        for k in range(S):
            s = k * beta
            for L in range(1, S - k + 1):
                e = s + L * beta
                pos = r * N * (N + 1) + s * (N + 1) + e
                if L == 1:
                    a = b = r * 64 + s
                else:
                    p = L.bit_length() - 1
                    a = _toff(r, p) + s
                    b = _toff(r, p) + s + L - (1 << p)
                vops.append((pos, a, b))
    for i in range((-len(vops)) % 16):
        vops.append((PLANE + i, 0, 0))
    return tab, vops


_TAB, _VOPS = _build_static()
_VN = len(_VOPS) // 16
_VPOS = np.array([o[0] for o in _VOPS], np.int32)
_VA = np.array([o[1] for o in _VOPS], np.int32)
_VB = np.array([o[2] for o in _VOPS], np.int32)

_NC, _NS = 2, 16
_NW = _NC * _NS
_PAIRS = 4 * 256
_PER_W = _PAIRS // _NW


def _sc_body(xr, vpos, va, vb, hid, msk, w, plane, mplane, vposv, vav, vbv):
    wid = lax.axis_index("s") * _NC + lax.axis_index("c")
    pltpu.sync_copy(vpos, vposv)
    pltpu.sync_copy(va, vav)
    pltpu.sync_copy(vb, vbv)

    zero = jnp.zeros((16,), jnp.float32)

    @pl.loop(0, PLANE_PAD // 16)
    def _(i):
        plane[pl.ds(i * 16, 16)] = zero
        mplane[pl.ds(i * 16, 16)] = zero

    one = jnp.full((16,), 1.0, jnp.float32)
    for v in range(_VN):
        plsc.store_scatter(mplane, [vposv[pl.ds(v * 16, 16)]], one)

    @pl.loop(0, _PER_W)
    def _(i):
        p = wid * _PER_W + i
        pltpu.sync_copy(xr.at[p], w.at[pl.ds(0, 256)])
        for dst, a, b in _TAB:
            w[pl.ds(dst, 16)] = jnp.maximum(w[pl.ds(a, 16)], w[pl.ds(b, 16)])
        for v in range(_VN):
            ia = vav[pl.ds(v * 16, 16)]
            ib = vbv[pl.ds(v * 16, 16)]
            ip = vposv[pl.ds(v * 16, 16)]
            vals = jnp.maximum(plsc.load_gather(w, [ia]),
                               plsc.load_gather(w, [ib]))
            plsc.store_scatter(plane, [ip], vals)
        pltpu.sync_copy(plane.at[pl.ds(0, PLANE)], hid.at[p])
        pltpu.sync_copy(mplane.at[pl.ds(0, PLANE)], msk.at[p])


@jax.jit
def _run(xr, vpos, va, vb):
    f = pl.kernel(
        _sc_body,
        out_type=(jax.ShapeDtypeStruct((_PAIRS, PLANE), jnp.float32),
                  jax.ShapeDtypeStruct((_PAIRS, PLANE), jnp.float32)),
        mesh=plsc.VectorSubcoreMesh(core_axis_name="c", subcore_axis_name="s"),
        compiler_params=pltpu.CompilerParams(needs_layout_passes=False),
        scratch_types=[
            pltpu.VMEM((W_SIZE,), jnp.float32),
            pltpu.VMEM((PLANE_PAD,), jnp.float32),
            pltpu.VMEM((PLANE_PAD,), jnp.float32),
            pltpu.VMEM((_VN * 16,), jnp.int32),
            pltpu.VMEM((_VN * 16,), jnp.int32),
            pltpu.VMEM((_VN * 16,), jnp.int32),
        ],
    )
    return f(xr, vpos, va, vb)


def kernel(x):
    B, ns, D, n = x.shape
    xr = x[:, :NSCALE].transpose(0, 2, 1, 3).reshape(B * D, NSCALE * n)
    hid, msk = _run(xr, jnp.asarray(_VPOS), jnp.asarray(_VA), jnp.asarray(_VB))
    shape = (B, D, NSCALE, n, n + 1)
    return hid.reshape(shape), msk.reshape(shape)


# speedup vs baseline: 1.1050x; 1.0557x over previous
import functools

import numpy as np
import jax
import jax.numpy as jnp
from jax import lax
from jax.experimental import pallas as pl
from jax.experimental.pallas import tpu as pltpu
from jax.experimental.pallas import tpu_sc as plsc

N = 64
NSCALE = 4
PLANE = NSCALE * N * (N + 1)
PLANE_PAD = PLANE + 16

def _toff(r, p):
    return 256 + (r * 6 + (p - 1)) * 80

W_SIZE = 256 + NSCALE * 6 * 80


def _build_static():
    tab = []
    for r in range(NSCALE):
        for p in range(1, (6 - r) + 1):
            length = 65 - (1 << p)
            prev = _toff(r, p - 1) if p > 1 else r * 64
            cur = _toff(r, p)
            h = 1 << (p - 1)
            for s0 in range(0, length, 16):
                tab.append((cur + s0, prev + s0, prev + s0 + h))
    vops = []
    for r in range(NSCALE):
        beta = 1 << r
        S = N >> r
        for k in range(S):
            s = k * beta
            for L in range(1, S - k + 1):
                e = s + L * beta
                pos = r * N * (N + 1) + s * (N + 1) + e
                if L == 1:
                    a = b = r * 64 + s
                else:
                    p = L.bit_length() - 1
                    a = _toff(r, p) + s
                    b = _toff(r, p) + s + L - (1 << p)
                vops.append((pos, a, b))
    for i in range((-len(vops)) % 16):
        vops.append((PLANE + i, 0, 0))
    return tab, vops


_TAB, _VOPS = _build_static()
_VN = len(_VOPS) // 16
_VPOS = np.array([o[0] for o in _VOPS], np.int32)
_VA = np.array([o[1] for o in _VOPS], np.int32)
_VB = np.array([o[2] for o in _VOPS], np.int32)

_NC, _NS = 2, 16
_NW = _NC * _NS
_PAIRS = 4 * 256
_PER_W = _PAIRS // _NW


def _sc_body(xr, vpos, va, vb, hid, msk,
             w, xin, plane0, plane1, mplane, vposv, vav, vbv,
             sem_h0, sem_h1, sem_m):
    wid = lax.axis_index("s") * _NC + lax.axis_index("c")
    base = wid * _PER_W
    pltpu.sync_copy(vpos, vposv)
    pltpu.sync_copy(va, vav)
    pltpu.sync_copy(vb, vbv)
    pltpu.sync_copy(xr.at[pl.ds(base * (NSCALE * N), _PER_W * (NSCALE * N))],
                    xin)

    zero = jnp.zeros((16,), jnp.float32)

    @pl.loop(0, PLANE_PAD // 16)
    def _(i):
        plane0[pl.ds(i * 16, 16)] = zero
        plane1[pl.ds(i * 16, 16)] = zero
        mplane[pl.ds(i * 16, 16)] = zero

    one = jnp.full((16,), 1.0, jnp.float32)
    for v in range(_VN):
        plsc.store_scatter(mplane, [vposv[pl.ds(v * 16, 16)]], one)

    planes = (plane0, plane1)
    sems = (sem_h0, sem_h1)

    @pl.loop(0, _PER_W // 2)
    def _(g):
        for b2 in range(2):
            i = g * 2 + b2
            p = base + i
            plane = planes[b2]
            sem = sems[b2]

            @pl.when(g > 0)
            def _():
                pltpu.make_async_copy(
                    plane.at[pl.ds(0, PLANE)], hid.at[p], sem).wait()

            xoff = i * (NSCALE * N)
            for c in range(16):
                w[pl.ds(c * 16, 16)] = xin[pl.ds(xoff + c * 16, 16)]
            for dst, a, b in _TAB:
                w[pl.ds(dst, 16)] = jnp.maximum(w[pl.ds(a, 16)],
                                                w[pl.ds(b, 16)])
            for v in range(_VN):
                ia = vav[pl.ds(v * 16, 16)]
                ib = vbv[pl.ds(v * 16, 16)]
                ip = vposv[pl.ds(v * 16, 16)]
                vals = jnp.maximum(plsc.load_gather(w, [ia]),
                                   plsc.load_gather(w, [ib]))
                plsc.store_scatter(plane, [ip], vals)

            pltpu.async_copy(plane.at[pl.ds(0, PLANE)], hid.at[p], sem)
            pltpu.async_copy(mplane.at[pl.ds(0, PLANE)], msk.at[p], sem_m)

            @pl.when(i >= 4)
            def _():
                pltpu.make_async_copy(
                    mplane.at[pl.ds(0, PLANE)], msk.at[p], sem_m).wait()

    for b2 in range(2):
        pltpu.make_async_copy(
            planes[b2].at[pl.ds(0, PLANE)], hid.at[base], sems[b2]).wait()
    for _ in range(4):
        pltpu.make_async_copy(
            mplane.at[pl.ds(0, PLANE)], msk.at[base], sem_m).wait()


@jax.jit
def _run(xr, vpos, va, vb):
    f = pl.kernel(
        _sc_body,
        out_type=(jax.ShapeDtypeStruct((_PAIRS, PLANE), jnp.float32),
                  jax.ShapeDtypeStruct((_PAIRS, PLANE), jnp.float32)),
        mesh=plsc.VectorSubcoreMesh(core_axis_name="c", subcore_axis_name="s"),
        compiler_params=pltpu.CompilerParams(needs_layout_passes=False),
        scratch_types=[
            pltpu.VMEM((W_SIZE,), jnp.float32),
            pltpu.VMEM((_PER_W * NSCALE * N,), jnp.float32),
            pltpu.VMEM((PLANE_PAD,), jnp.float32),
            pltpu.VMEM((PLANE_PAD,), jnp.float32),
            pltpu.VMEM((PLANE_PAD,), jnp.float32),
            pltpu.VMEM((_VN * 16,), jnp.int32),
            pltpu.VMEM((_VN * 16,), jnp.int32),
            pltpu.VMEM((_VN * 16,), jnp.int32),
            pltpu.SemaphoreType.DMA,
            pltpu.SemaphoreType.DMA,
            pltpu.SemaphoreType.DMA,
        ],
    )
    return f(xr, vpos, va, vb)


def kernel(x):
    B, ns, D, n = x.shape
    xr = x[:, :NSCALE].transpose(0, 2, 1, 3).reshape(B * D * NSCALE * n)
    hid, msk = _run(xr, jnp.asarray(_VPOS), jnp.asarray(_VA), jnp.asarray(_VB))
    shape = (B, D, NSCALE, n, n + 1)
    return hid.reshape(shape), msk.reshape(shape)


# speedup vs baseline: 1.6059x; 1.4533x over previous
import functools

import numpy as np
import jax
import jax.numpy as jnp
from jax import lax
from jax.experimental import pallas as pl
from jax.experimental.pallas import tpu as pltpu
from jax.experimental.pallas import tpu_sc as plsc

N = 64
NSCALE = 4

def _toff(r, p):
    return 256 + (r * 6 + (p - 1)) * 80

W_SIZE = 256 + NSCALE * 6 * 80


def _build_static():
    tab = []
    for r in range(NSCALE):
        for p in range(1, (6 - r) + 1):
            length = 65 - (1 << p)
            prev = _toff(r, p - 1) if p > 1 else r * 64
            cur = _toff(r, p)
            h = 1 << (p - 1)
            for s0 in range(0, length, 16):
                tab.append((cur + s0, prev + s0, prev + s0 + h))
    vops = []
    for r in range(NSCALE):
        beta = 1 << r
        S = N >> r
        for k in range(S):
            s = k * beta
            for L in range(1, S - k + 1):
                e = s + L * beta
                if L == 1:
                    a = b = r * 64 + s
                else:
                    p = L.bit_length() - 1
                    a = _toff(r, p) + s
                    b = _toff(r, p) + s + L - (1 << p)
                vops.append((r, s, e, a, b))
    while len(vops) % 16:
        c0 = (len(vops) // 16) * 16
        vops.append(vops[c0])
    return tab, vops


_TAB, _VOPS = _build_static()
_VN = len(_VOPS) // 16
_IDX_ROWS = (_VN * 16 + 127) // 128 + 1


def _pad_rows(vals):
    out = np.zeros((_IDX_ROWS * 128,), np.int32)
    out[:len(vals)] = vals
    return out.reshape(_IDX_ROWS, 128)


_VR = _pad_rows([o[0] for o in _VOPS])
_VS = _pad_rows([o[1] for o in _VOPS])
_VE = _pad_rows([o[2] for o in _VOPS])
_VA = _pad_rows([o[3] for o in _VOPS])
_VB = _pad_rows([o[4] for o in _VOPS])

_NC, _NS = 2, 16
_NW = _NC * _NS
_B, _D = 4, 256
_PAIRS = _B * _D
_PER_W = _PAIRS // _NW


def _idx_vec(ref, v):
    return ref[v // 8, pl.ds((v % 8) * 16, 16)]


def _sc_body(xr, vr, vs, ve, va, vb, hid, msk,
             w, stage, plane0, plane1, mplane,
             vrv, vsv, vev, vav, vbv,
             sem_h0, sem_h1, sem_m):
    wid = lax.axis_index("s") * _NC + lax.axis_index("c")
    base = wid * _PER_W
    pltpu.sync_copy(vr, vrv)
    pltpu.sync_copy(vs, vsv)
    pltpu.sync_copy(ve, vev)
    pltpu.sync_copy(va, vav)
    pltpu.sync_copy(vb, vbv)

    zero = jnp.zeros((16,), jnp.float32)

    @pl.loop(0, N)
    def _(s):
        for r in range(NSCALE):
            for c in (0, 16, 32, 48, 49):
                plane0[r, s, pl.ds(c, 16)] = zero
                plane1[r, s, pl.ds(c, 16)] = zero
                mplane[r, s, pl.ds(c, 16)] = zero

    one = jnp.full((16,), 1.0, jnp.float32)
    for v in range(_VN):
        plsc.store_scatter(
            mplane,
            [_idx_vec(vrv, v), _idx_vec(vsv, v), _idx_vec(vev, v)],
            one)

    planes = (plane0, plane1)
    sems = (sem_h0, sem_h1)

    @pl.loop(0, _PER_W // 2)
    def _(g):
        for b2 in range(2):
            i = g * 2 + b2
            p = base + i
            bi = p // _D
            di = p % _D
            plane = planes[b2]
            sem = sems[b2]

            @pl.when(g > 0)
            def _():
                pltpu.make_async_copy(plane, hid.at[bi, di], sem).wait()

            pltpu.sync_copy(xr.at[pl.ds(2 * p, 2)], stage)
            for c in range(16):
                w[pl.ds(c * 16, 16)] = stage[c // 8, pl.ds((c % 8) * 16, 16)]
            for dst, a, b in _TAB:
                w[pl.ds(dst, 16)] = jnp.maximum(w[pl.ds(a, 16)],
                                                w[pl.ds(b, 16)])
            for v in range(_VN):
                vals = jnp.maximum(
                    plsc.load_gather(w, [_idx_vec(vav, v)]),
                    plsc.load_gather(w, [_idx_vec(vbv, v)]))
                plsc.store_scatter(
                    plane,
                    [_idx_vec(vrv, v), _idx_vec(vsv, v), _idx_vec(vev, v)],
                    vals)

            pltpu.async_copy(plane, hid.at[bi, di], sem)
            pltpu.async_copy(mplane, msk.at[bi, di], sem_m)

            @pl.when(i >= 4)
            def _():
                pltpu.make_async_copy(mplane, msk.at[bi, di], sem_m).wait()

    bi0 = base // _D
    di0 = base % _D
    for b2 in range(2):
        pltpu.make_async_copy(planes[b2], hid.at[bi0, di0], sems[b2]).wait()
    for _ in range(4):
        pltpu.make_async_copy(mplane, msk.at[bi0, di0], sem_m).wait()


@jax.jit
def _run(xr, vr, vs, ve, va, vb):
    f = pl.kernel(
        _sc_body,
        out_type=(jax.ShapeDtypeStruct((_B, _D, NSCALE, N, N + 1),
                                       jnp.float32),
                  jax.ShapeDtypeStruct((_B, _D, NSCALE, N, N + 1),
                                       jnp.float32)),
        mesh=plsc.VectorSubcoreMesh(core_axis_name="c", subcore_axis_name="s"),
        compiler_params=pltpu.CompilerParams(needs_layout_passes=False,
                                             use_tc_tiling_on_sc=True),
        scratch_types=[
            pltpu.VMEM((W_SIZE,), jnp.float32),
            pltpu.VMEM((2, 128), jnp.float32),
            pltpu.VMEM((NSCALE, N, N + 1), jnp.float32),
            pltpu.VMEM((NSCALE, N, N + 1), jnp.float32),
            pltpu.VMEM((NSCALE, N, N + 1), jnp.float32),
            pltpu.VMEM((_IDX_ROWS, 128), jnp.int32),
            pltpu.VMEM((_IDX_ROWS, 128), jnp.int32),
            pltpu.VMEM((_IDX_ROWS, 128), jnp.int32),
            pltpu.VMEM((_IDX_ROWS, 128), jnp.int32),
            pltpu.VMEM((_IDX_ROWS, 128), jnp.int32),
            pltpu.SemaphoreType.DMA,
            pltpu.SemaphoreType.DMA,
            pltpu.SemaphoreType.DMA,
        ],
    )
    return f(xr, vr, vs, ve, va, vb)


def kernel(x):
    B, ns, D, n = x.shape
    xr = x[:, :NSCALE].transpose(0, 2, 1, 3).reshape(B * D * NSCALE * n // 128,
                                                     128)
    return _run(xr, jnp.asarray(_VR), jnp.asarray(_VS), jnp.asarray(_VE),
                jnp.asarray(_VA), jnp.asarray(_VB))
